# baseline (speedup 1.0000x reference)
import functools

import jax
import jax.numpy as jnp
from jax import lax
from jax.experimental import pallas as pl
from jax.experimental.pallas import tpu as pltpu
from jax.experimental.pallas import tpu_sc as plsc

_B = 128
_N = 2048
_L = 16
_NC = 2
_NS = 16
_NW = _NC * _NS
_RPW = _B // _NW
_CH = _N // _L
_U = 8

_NEG = float("-inf")


def _spa_body(x_hbm, out_hbm, xin, sem_in, sem_out):
    wid = lax.axis_index("s") * _NC + lax.axis_index("c")
    base = wid * _RPW
    lanes = lax.iota(jnp.int32, _L)

    for r in range(_RPW):
        pltpu.async_copy(
            x_hbm.at[pl.ds(base + r, 1)], xin.at[pl.ds(r, 1)], sem_in)

    def row_body(r, _):
        pltpu.make_async_copy(
            x_hbm.at[pl.ds(base + r, 1)], xin.at[pl.ds(r, 1)],
            sem_in).wait()

        def scan_body(c, carry):
            l1, l2 = carry
            for k in range(_U):
                v = xin[r, pl.ds((c * _U + k) * _L, _L)]
                l2 = jnp.maximum(l2, jnp.minimum(l1, v))
                l1 = jnp.maximum(l1, v)
            return l1, l2

        l1, l2 = lax.fori_loop(
            0, _CH // _U, scan_body,
            (jnp.full((_L,), _NEG), jnp.full((_L,), _NEG)))

        s1, _ = plsc.sort_key_val(l1, lanes, descending=True)
        m1 = s1[0]
        m2 = jnp.maximum(s1[1], jnp.max(l2))
        m1v = jnp.full((_L,), m1)
        p1v = jnp.full((_L,), jnp.maximum(m1, jnp.float32(0.0)))
        p2v = jnp.full((_L,), jnp.maximum(m2, jnp.float32(0.0)))

        def fill_body(c, _):
            for k in range(_U):
                v = xin[r, pl.ds((c * _U + k) * _L, _L)]
                xin[r, pl.ds((c * _U + k) * _L, _L)] = jnp.where(
                    v == m1v, p2v, p1v)
            return 0

        lax.fori_loop(0, _CH // _U, fill_body, 0)

        pltpu.async_copy(
            xin.at[pl.ds(r, 1)], out_hbm.at[pl.ds(base + r, 1)], sem_out)
        return 0

    lax.fori_loop(0, _RPW, row_body, 0)

    for r in range(_RPW):
        pltpu.make_async_copy(
            xin.at[pl.ds(r, 1)], out_hbm.at[pl.ds(base + r, 1)],
            sem_out).wait()


_spa_payment = functools.partial(
    pl.kernel,
    out_type=jax.ShapeDtypeStruct((_B, _N), jnp.float32),
    mesh=plsc.VectorSubcoreMesh(core_axis_name="c", subcore_axis_name="s"),
    scratch_types=[
        pltpu.VMEM((_RPW, _N), jnp.float32),
        pltpu.SemaphoreType.DMA,
        pltpu.SemaphoreType.DMA,
    ],
    compiler_params=pltpu.CompilerParams(needs_layout_passes=False),
)(_spa_body)


def kernel(x):
    return _spa_payment(x)


# speedup vs baseline: 1.0503x; 1.0503x over previous
import functools

import jax
import jax.numpy as jnp
from jax import lax
from jax.experimental import pallas as pl
from jax.experimental.pallas import tpu as pltpu
from jax.experimental.pallas import tpu_sc as plsc

_B = 128
_N = 2048
_L = 16
_NC = 2
_NS = 16
_NW = _NC * _NS
_RPW = _B // _NW
_CH = _N // _L
_U = 8

_NEG = float("-inf")


def _spa_body(x_hbm, out_hbm, xin, sem_in, sem_out):
    wid = lax.axis_index("s") * _NC + lax.axis_index("c")
    base = wid * _RPW
    lanes = lax.iota(jnp.int32, _L)

    in_cp = [
        pltpu.async_copy(
            x_hbm.at[pl.ds(base + 2 * p, 2)], xin.at[pl.ds(2 * p, 2)],
            sem_in)
        for p in range(_RPW // 2)
    ]
    out_cp = []
    for p in range(_RPW // 2):
        ra, rb = 2 * p, 2 * p + 1
        in_cp[p].wait()

        def scan_body(c, carry):
            l1a, l2a, l1b, l2b = carry
            for k in range(_U):
                off = (c * _U + k) * _L
                va = xin[ra, pl.ds(off, _L)]
                vb = xin[rb, pl.ds(off, _L)]
                l2a = jnp.maximum(l2a, jnp.minimum(l1a, va))
                l1a = jnp.maximum(l1a, va)
                l2b = jnp.maximum(l2b, jnp.minimum(l1b, vb))
                l1b = jnp.maximum(l1b, vb)
            return l1a, l2a, l1b, l2b

        neg = jnp.full((_L,), _NEG)
        l1a, l2a, l1b, l2b = lax.fori_loop(
            0, _CH // _U, scan_body, (neg, neg, neg, neg))

        s1a, _ = plsc.sort_key_val(l1a, lanes, descending=True)
        s1b, _ = plsc.sort_key_val(l1b, lanes, descending=True)
        m1a, m1b = s1a[0], s1b[0]
        m2a = jnp.maximum(s1a[1], jnp.max(l2a))
        m2b = jnp.maximum(s1b[1], jnp.max(l2b))
        m1va = jnp.full((_L,), m1a)
        m1vb = jnp.full((_L,), m1b)
        p1va = jnp.full((_L,), jnp.maximum(m1a, jnp.float32(0.0)))
        p2va = jnp.full((_L,), jnp.maximum(m2a, jnp.float32(0.0)))
        p1vb = jnp.full((_L,), jnp.maximum(m1b, jnp.float32(0.0)))
        p2vb = jnp.full((_L,), jnp.maximum(m2b, jnp.float32(0.0)))

        def fill_body(c, _):
            for k in range(_U):
                off = (c * _U + k) * _L
                va = xin[ra, pl.ds(off, _L)]
                vb = xin[rb, pl.ds(off, _L)]
                xin[ra, pl.ds(off, _L)] = jnp.where(va == m1va, p2va, p1va)
                xin[rb, pl.ds(off, _L)] = jnp.where(vb == m1vb, p2vb, p1vb)
            return 0

        lax.fori_loop(0, _CH // _U, fill_body, 0)

        out_cp.append(
            pltpu.async_copy(
                xin.at[pl.ds(2 * p, 2)], out_hbm.at[pl.ds(base + 2 * p, 2)],
                sem_out))
    for cp in out_cp:
        cp.wait()


_spa_payment = functools.partial(
    pl.kernel,
    out_type=jax.ShapeDtypeStruct((_B, _N), jnp.float32),
    mesh=plsc.VectorSubcoreMesh(core_axis_name="c", subcore_axis_name="s"),
    scratch_types=[
        pltpu.VMEM((_RPW, _N), jnp.float32),
        pltpu.SemaphoreType.DMA,
        pltpu.SemaphoreType.DMA,
    ],
    compiler_params=pltpu.CompilerParams(needs_layout_passes=False),
)(_spa_body)


def kernel(x):
    return _spa_payment(x)


# speedup vs baseline: 1.0520x; 1.0016x over previous
import functools

import jax
import jax.numpy as jnp
from jax import lax
from jax.experimental import pallas as pl
from jax.experimental.pallas import tpu as pltpu
from jax.experimental.pallas import tpu_sc as plsc

_B = 128
_N = 2048
_L = 16
_NC = 2
_NS = 16
_NW = _NC * _NS
_RPW = _B // _NW
_CH = _N // _L
_U = 8

_NEG = float("-inf")


def _spa_body(x_hbm, out_hbm, xin, sem_in, sem_out):
    wid = lax.axis_index("s") * _NC + lax.axis_index("c")
    base = wid * _RPW
    lanes = lax.iota(jnp.int32, _L)

    in_cp = pltpu.async_copy(x_hbm.at[pl.ds(base, _RPW)], xin, sem_in)
    in_cp.wait()

    def scan_body(c, carry):
        l1s, l2s = carry
        for k in range(_U):
            off = (c * _U + k) * _L
            vs = [xin[r, pl.ds(off, _L)] for r in range(_RPW)]
            l2s = tuple(
                jnp.maximum(l2s[r], jnp.minimum(l1s[r], vs[r]))
                for r in range(_RPW))
            l1s = tuple(
                jnp.maximum(l1s[r], vs[r]) for r in range(_RPW))
        return l1s, l2s

    neg = jnp.full((_L,), _NEG)
    l1s, l2s = lax.fori_loop(
        0, _CH // _U, scan_body,
        ((neg,) * _RPW, (neg,) * _RPW))

    m1v, p1v, p2v = [], [], []
    for r in range(_RPW):
        s1, _ = plsc.sort_key_val(l1s[r], lanes, descending=True)
        m1 = s1[0]
        m2 = jnp.maximum(s1[1], jnp.max(l2s[r]))
        m1v.append(jnp.full((_L,), m1))
        p1v.append(jnp.full((_L,), jnp.maximum(m1, jnp.float32(0.0))))
        p2v.append(jnp.full((_L,), jnp.maximum(m2, jnp.float32(0.0))))

    out_cp = []
    for p in range(_RPW // 2):
        ra, rb = 2 * p, 2 * p + 1

        def fill_body(c, _):
            for k in range(_U):
                off = (c * _U + k) * _L
                va = xin[ra, pl.ds(off, _L)]
                vb = xin[rb, pl.ds(off, _L)]
                xin[ra, pl.ds(off, _L)] = jnp.where(
                    va == m1v[ra], p2v[ra], p1v[ra])
                xin[rb, pl.ds(off, _L)] = jnp.where(
                    vb == m1v[rb], p2v[rb], p1v[rb])
            return 0

        lax.fori_loop(0, _CH // _U, fill_body, 0)

        out_cp.append(
            pltpu.async_copy(
                xin.at[pl.ds(2 * p, 2)], out_hbm.at[pl.ds(base + 2 * p, 2)],
                sem_out))
    for cp in out_cp:
        cp.wait()


_spa_payment = functools.partial(
    pl.kernel,
    out_type=jax.ShapeDtypeStruct((_B, _N), jnp.float32),
    mesh=plsc.VectorSubcoreMesh(core_axis_name="c", subcore_axis_name="s"),
    scratch_types=[
        pltpu.VMEM((_RPW, _N), jnp.float32),
        pltpu.SemaphoreType.DMA,
        pltpu.SemaphoreType.DMA,
    ],
    compiler_params=pltpu.CompilerParams(needs_layout_passes=False),
)(_spa_body)


def kernel(x):
    return _spa_payment(x)
